# Initial kernel scaffold; baseline (speedup 1.0000x reference)
import jax
import jax.numpy as jnp
from jax.experimental import pallas as pl


def kernel(x, edge_index, W):
    raise NotImplementedError("write your pallas kernel here")


# speedup vs baseline: 13.5993x; 13.5993x over previous
import dataclasses

import jax
import jax.numpy as jnp
from jax import lax
from jax.experimental import pallas as pl
from jax.experimental.pallas import tpu as pltpu
from jax.experimental.pallas import tpu_sc as plsc

N = 10000
D = 128
NROWS = 10240
TRASH = N
NC, NS = 2, 16
NW = NC * NS
B = 128
RPS = NROWS // NS
f32 = jnp.float32


def _z():
    return jnp.int32(0)


_mesh = plsc.VectorSubcoreMesh(core_axis_name="c", subcore_axis_name="s")

_sc_params = pltpu.CompilerParams()
if "needs_layout_passes" in pltpu.CompilerParams.__dataclass_fields__:
    _sc_params = dataclasses.replace(_sc_params, needs_layout_passes=False)


def _sc_degree(dst_pad, ep):
    epw = ep // NW
    nb = epw // B

    @pl.kernel(out_type=jax.ShapeDtypeStruct((NW, NROWS), f32),
               mesh=_mesh,
               compiler_params=_sc_params,
               scratch_types=[pltpu.VMEM((B,), jnp.int32),
                              pltpu.VMEM((NROWS,), f32)])
    def deg_kernel(dst_hbm, out_hbm, idx_v, deg_v):
        cid = lax.axis_index("c").astype(jnp.int32)
        sid = lax.axis_index("s").astype(jnp.int32)
        wid = cid * jnp.int32(NS) + sid
        zeros16 = jnp.zeros((16,), f32)
        ones16 = jnp.ones((16,), f32)

        @pl.loop(jnp.int32(0), jnp.int32(NROWS // 16))
        def _(i):
            i = jnp.asarray(i, jnp.int32)
            deg_v[pl.ds(i * jnp.int32(16), 16)] = zeros16

        base = wid * jnp.int32(epw)

        @pl.loop(jnp.int32(0), jnp.int32(nb))
        def _(b):
            b = jnp.asarray(b, jnp.int32)
            pltpu.sync_copy(dst_hbm.at[pl.ds(base + b * jnp.int32(B), B)],
                            idx_v)
            for j in range(B // 16):
                idx = idx_v[pl.ds(j * 16, 16)]
                plsc.addupdate_scatter(deg_v, [idx], ones16)

        pltpu.sync_copy(deg_v, out_hbm.at[wid])

    return deg_kernel(dst_pad)


def _sc_edge_agg(hs, src_pad, dst_pad, zerosD, ep):
    eps = ep // NC
    epw = eps // NS
    nb = epw // B

    @pl.kernel(out_type=jax.ShapeDtypeStruct((NC, NROWS, D), f32),
               mesh=_mesh,
               scratch_types=[pltpu.VMEM((B,), jnp.int32),
                              pltpu.VMEM((B,), jnp.int32),
                              pltpu.VMEM((B, D), f32),
                              pltpu.VMEM_SHARED((NROWS, D), f32)])
    def agg_kernel(hs_hbm, src_hbm, dst_hbm, zeros_hbm, out_hbm,
                   src_v, dst_v, rows_v, acc_sh):
        cid = lax.axis_index("c").astype(jnp.int32)
        sid = lax.axis_index("s").astype(jnp.int32)
        pltpu.sync_copy(zeros_hbm, acc_sh.at[pl.ds(sid * jnp.int32(RPS), RPS)])
        plsc.subcore_barrier()
        base = cid * jnp.int32(eps) + sid * jnp.int32(epw)

        @pl.loop(jnp.int32(0), jnp.int32(nb))
        def _(b):
            b = jnp.asarray(b, jnp.int32)
            off = base + b * jnp.int32(B)
            pltpu.sync_copy(src_hbm.at[pl.ds(off, B)], src_v)
            pltpu.sync_copy(dst_hbm.at[pl.ds(off, B)], dst_v)
            pltpu.sync_copy(hs_hbm.at[src_v], rows_v)
            pltpu.sync_copy(rows_v, acc_sh.at[dst_v], add=True)

        plsc.subcore_barrier()
        pltpu.sync_copy(acc_sh.at[pl.ds(sid * jnp.int32(RPS), RPS)],
                        out_hbm.at[cid, pl.ds(sid * jnp.int32(RPS), RPS)])

    return agg_kernel(hs, src_pad, dst_pad, zerosD)


def _tc_prep(x_pad, W, cnt):
    RB = 1024

    def body(x_ref, w_ref, cnt_ref, hs_ref, hself_ref):
        h = lax.dot(x_ref[...], w_ref[...],
                    precision=lax.Precision.HIGHEST)
        deg = jnp.sum(cnt_ref[...], axis=0)[:, None] + 1.0
        hs_ref[...] = h * lax.rsqrt(deg)
        hself_ref[...] = h / deg

    return pl.pallas_call(
        body,
        grid=(NROWS // RB,),
        in_specs=[pl.BlockSpec((RB, D), lambda i: (i, _z())),
                  pl.BlockSpec((D, D), lambda i: (_z(), _z())),
                  pl.BlockSpec((NW, RB), lambda i: (_z(), i))],
        out_specs=[pl.BlockSpec((RB, D), lambda i: (i, _z())),
                   pl.BlockSpec((RB, D), lambda i: (i, _z()))],
        out_shape=[jax.ShapeDtypeStruct((NROWS, D), f32),
                   jax.ShapeDtypeStruct((NROWS, D), f32)],
    )(x_pad, W, cnt)


def _tc_final(accs, cnt, hself):
    RB = 1024

    def body(acc_ref, cnt_ref, hself_ref, out_ref):
        deg = jnp.sum(cnt_ref[...], axis=0)[:, None] + 1.0
        out_ref[...] = ((acc_ref[0] + acc_ref[1]) * lax.rsqrt(deg)
                        + hself_ref[...])

    return pl.pallas_call(
        body,
        grid=(NROWS // RB,),
        in_specs=[pl.BlockSpec((NC, RB, D), lambda i: (_z(), i, _z())),
                  pl.BlockSpec((NW, RB), lambda i: (_z(), i)),
                  pl.BlockSpec((RB, D), lambda i: (i, _z()))],
        out_specs=pl.BlockSpec((RB, D), lambda i: (i, _z())),
        out_shape=jax.ShapeDtypeStruct((NROWS, D), f32),
    )(accs, cnt, hself)


def kernel(x, edge_index, W):
    src = edge_index[0].astype(jnp.int32)
    dst = edge_index[1].astype(jnp.int32)
    e = src.shape[0]
    chunk = NW * B
    ep = ((e + chunk - 1) // chunk) * chunk
    pad = ep - e
    if pad:
        src = jnp.concatenate([src, jnp.full((pad,), TRASH, jnp.int32)])
        dst = jnp.concatenate([dst, jnp.full((pad,), TRASH, jnp.int32)])
    x_pad = jnp.pad(x.astype(f32), ((0, NROWS - N), (0, 0)))
    zerosD = jnp.zeros((RPS, D), f32)

    cnt = _sc_degree(dst, ep)
    hs, hself = _tc_prep(x_pad, W.astype(f32), cnt)
    accs = _sc_edge_agg(hs, src, dst, zerosD, ep)
    out = _tc_final(accs, cnt, hself)
    return out[:N]
